# baseline (speedup 1.0000x reference)
import functools

import jax
import jax.numpy as jnp
from jax.experimental import pallas as pl
from jax.experimental.pallas import tpu as pltpu


def _gate_kernel(x_ref, w1_ref, b1_ref, w2_ref, b2_ref,
                 out_ref, scale_ref, *, inv_hw, bb):
    x = x_ref[...]

    avg = jnp.sum(x, axis=-1) * inv_hw
    mx = jnp.max(x, axis=-1)
    pooled = jnp.concatenate([avg.T, mx.T], axis=-1)

    h = jnp.dot(w1_ref[...], pooled,
                preferred_element_type=jnp.float32) + b1_ref[...]
    h = jnp.maximum(h, 0.0)
    att = jnp.dot(w2_ref[...], h,
                  preferred_element_type=jnp.float32) + b2_ref[...]

    att_sum = att[:, :bb] + att[:, bb:]
    scale = jax.nn.sigmoid(att_sum).T[:, :, None]

    out_ref[...] = x * scale
    scale_ref[...] = scale


def kernel(x, w1, b1, w2, b2):
    B, C, H, W = x.shape
    HW = H * W
    hidden = w1.shape[0]

    x_flat = x.reshape(B, C, HW)
    b1_2d = b1.reshape(hidden, 1)
    b2_2d = b2.reshape(C, 1)

    ch = 8 if B % 8 == 0 else B
    bb = 2 if ch % 2 == 0 else 1

    gate_call = pl.pallas_call(
        functools.partial(_gate_kernel, inv_hw=1.0 / HW, bb=bb),
        out_shape=(
            jax.ShapeDtypeStruct((ch, C, HW), jnp.float32),
            jax.ShapeDtypeStruct((ch, C, 1), jnp.float32),
        ),
        grid=(ch // bb,),
        in_specs=[
            pl.BlockSpec((bb, C, HW), lambda b: (b, 0, 0)),
            pl.BlockSpec((hidden, C), lambda b: (0, 0)),
            pl.BlockSpec((hidden, 1), lambda b: (0, 0)),
            pl.BlockSpec((C, hidden), lambda b: (0, 0)),
            pl.BlockSpec((C, 1), lambda b: (0, 0)),
        ],
        out_specs=(
            pl.BlockSpec((bb, C, HW), lambda b: (b, 0, 0)),
            pl.BlockSpec((bb, C, 1), lambda b: (b, 0, 0)),
        ),
        compiler_params=pltpu.CompilerParams(
            dimension_semantics=("arbitrary",),
            vmem_limit_bytes=24 * 1024 * 1024),
    )

    outs, scales = [], []
    for k in range(B // ch):
        xk = jax.lax.optimization_barrier(
            jax.lax.slice_in_dim(x_flat, k * ch, (k + 1) * ch, axis=0))
        o, s = gate_call(xk, w1, b1_2d, w2, b2_2d)
        outs.append(o)
        scales.append(s)

    out_flat = jnp.concatenate(outs, axis=0) if len(outs) > 1 else outs[0]
    scale_flat = jnp.concatenate(scales, axis=0) if len(scales) > 1 else scales[0]

    scale_full = jnp.broadcast_to(scale_flat.reshape(B, C, 1, 1), (B, C, H, W))
    return (out_flat.reshape(B, C, H, W), scale_full)
